# baseline (speedup 1.0000x reference)
import functools

import jax
import jax.numpy as jnp
from jax import lax
from jax.experimental import pallas as pl
from jax.experimental.pallas import tpu as pltpu
from jax.experimental.pallas import tpu_sc as plsc

_Q = 0.7
_K = 0.8
_C = (1.0 - _K**_Q) / _Q

_B = 16
_NCLS = 21
_H = 256
_W = 256
_ROWS = 2048
_N = _B * _H * _W

_NC = 2
_NS = 16
_NW = _NC * _NS
_HSL = _H // _NW


def _sc_gather_body(table_ref, idx_ref, out_ref, idx_v, buf_v, sem_a, sem_b):
    wid = lax.axis_index("s") * _NC + lax.axis_index("c")
    pltpu.sync_copy(idx_ref, idx_v)
    h0 = wid * _HSL
    ga = pltpu.async_copy(
        table_ref.at[idx_v.at[pl.ds(0, _B // 2)], pl.ds(h0, _HSL)],
        buf_v.at[pl.ds(0, _B // 2)], sem_a)
    gb = pltpu.async_copy(
        table_ref.at[idx_v.at[pl.ds(_B // 2, _B // 2)], pl.ds(h0, _HSL)],
        buf_v.at[pl.ds(_B // 2, _B // 2)], sem_b)
    ga.wait()
    wa = pltpu.async_copy(buf_v.at[pl.ds(0, _B // 2)],
                          out_ref.at[pl.ds(0, _B // 2), pl.ds(h0, _HSL)], sem_a)
    gb.wait()
    wb = pltpu.async_copy(buf_v.at[pl.ds(_B // 2, _B // 2)],
                          out_ref.at[pl.ds(_B // 2, _B // 2), pl.ds(h0, _HSL)],
                          sem_b)
    wa.wait()
    wb.wait()


@jax.jit
def _sc_gather(table, idx):
    mesh = plsc.VectorSubcoreMesh(
        core_axis_name="c", subcore_axis_name="s",
        num_cores=_NC, num_subcores=_NS)
    return pl.kernel(
        _sc_gather_body,
        out_type=jax.ShapeDtypeStruct((_B, _H, _W), jnp.float32),
        mesh=mesh,
        scratch_types=[
            pltpu.VMEM((16,), jnp.int32),
            pltpu.VMEM((_B, _HSL, _W), jnp.float32),
            pltpu.SemaphoreType.DMA,
            pltpu.SemaphoreType.DMA,
        ],
    )(table, idx)


def _tc_term_body(logits_ref, targets_ref, t_ref):
    l = logits_ref[...]
    t = targets_ref[...]
    m = jnp.max(l, axis=1)
    e = jnp.exp(l - m[:, None])
    s = jnp.sum(e, axis=1)
    cls = lax.broadcasted_iota(jnp.int32, l.shape, 1)
    lt = jnp.sum(jnp.where(cls == t[:, None], l, 0.0), axis=1)
    log_yg = (lt - m) - jnp.log(s)
    pow_q = jnp.exp(_Q * log_yg)
    t_ref[...] = ((1.0 - pow_q) * (1.0 / _Q) - _C) * (1.0 / _N)


@functools.partial(jax.jit, static_argnames=("block_b",))
def _tc_term(logits, targets, block_b=2):
    nsteps = _B // block_b
    return pl.pallas_call(
        _tc_term_body,
        grid=(nsteps,),
        in_specs=[
            pl.BlockSpec((block_b, _NCLS, _H, _W), lambda b: (b, 0, 0, 0)),
            pl.BlockSpec((block_b, _H, _W), lambda b: (b, 0, 0)),
        ],
        out_specs=pl.BlockSpec((block_b, _H, _W), lambda b: (b, 0, 0)),
        out_shape=jax.ShapeDtypeStruct((_B, _H, _W), jnp.float32),
    )(logits, targets)


def _tc_dot_body(t_ref, w_ref, out_ref):
    step = pl.program_id(0)

    @pl.when(step == 0)
    def _init():
        out_ref[0, 0] = 0.0

    out_ref[0, 0] += jnp.sum(t_ref[...] * w_ref[...])


@functools.partial(jax.jit, static_argnames=("block_b",))
def _tc_dot(t16, w16, block_b=4):
    nsteps = _B // block_b
    return pl.pallas_call(
        _tc_dot_body,
        grid=(nsteps,),
        in_specs=[
            pl.BlockSpec((block_b, _H, _W), lambda b: (b, 0, 0)),
            pl.BlockSpec((block_b, _H, _W), lambda b: (b, 0, 0)),
        ],
        out_specs=pl.BlockSpec((1, 1), lambda b: (0, 0),
                               memory_space=pltpu.SMEM),
        out_shape=jax.ShapeDtypeStruct((1, 1), jnp.float32),
    )(t16, w16)


def kernel(logits, weight, targets, indexes):
    w16 = _sc_gather(weight.reshape(_ROWS, _H, _W), indexes)
    t16 = _tc_term(logits, targets.reshape(_B, _H, _W))
    out = _tc_dot(t16, w16)
    return out[0, 0]


# speedup vs baseline: 1.0277x; 1.0277x over previous
import functools

import jax
import jax.numpy as jnp
from jax import lax
from jax.experimental import pallas as pl
from jax.experimental.pallas import tpu as pltpu
from jax.experimental.pallas import tpu_sc as plsc

_Q = 0.7
_K = 0.8
_C = (1.0 - _K**_Q) / _Q

_B = 16
_NCLS = 21
_H = 256
_W = 256
_ROWS = 2048
_N = _B * _H * _W

_NC = 2
_NS = 16
_NW = _NC * _NS
_HSL = _H // _NW


def _sc_gather_body(table_ref, idx_ref, out_ref, idx_v, buf_v, sem_a, sem_b):
    wid = lax.axis_index("s") * _NC + lax.axis_index("c")
    pltpu.sync_copy(idx_ref, idx_v)
    h0 = wid * _HSL
    ga = pltpu.async_copy(
        table_ref.at[idx_v.at[pl.ds(0, _B // 2)], pl.ds(h0, _HSL)],
        buf_v.at[pl.ds(0, _B // 2)], sem_a)
    gb = pltpu.async_copy(
        table_ref.at[idx_v.at[pl.ds(_B // 2, _B // 2)], pl.ds(h0, _HSL)],
        buf_v.at[pl.ds(_B // 2, _B // 2)], sem_b)
    ga.wait()
    wa = pltpu.async_copy(buf_v.at[pl.ds(0, _B // 2)],
                          out_ref.at[pl.ds(0, _B // 2), pl.ds(h0, _HSL)], sem_a)
    gb.wait()
    wb = pltpu.async_copy(buf_v.at[pl.ds(_B // 2, _B // 2)],
                          out_ref.at[pl.ds(_B // 2, _B // 2), pl.ds(h0, _HSL)],
                          sem_b)
    wa.wait()
    wb.wait()


@jax.jit
def _sc_gather(table, idx):
    mesh = plsc.VectorSubcoreMesh(
        core_axis_name="c", subcore_axis_name="s",
        num_cores=_NC, num_subcores=_NS)
    return pl.kernel(
        _sc_gather_body,
        out_type=jax.ShapeDtypeStruct((_B, _H, _W), jnp.float32),
        mesh=mesh,
        scratch_types=[
            pltpu.VMEM((16,), jnp.int32),
            pltpu.VMEM((_B, _HSL, _W), jnp.float32),
            pltpu.SemaphoreType.DMA,
            pltpu.SemaphoreType.DMA,
        ],
    )(table, idx)


def _tc_term_body(logits_ref, targets_ref, t_ref):
    l = logits_ref[...]
    t = targets_ref[...]
    m = jnp.max(l, axis=1)
    e = jnp.exp(l - m[:, None])
    s = jnp.sum(e, axis=1)
    cls = lax.broadcasted_iota(jnp.int32, l.shape, 1)
    lt = jnp.sum(jnp.where(cls == t[:, None], l, 0.0), axis=1)
    log_yg = (lt - m) - jnp.log(s)
    pow_q = jnp.exp(_Q * log_yg)
    t_ref[...] = (((1.0 - pow_q) * (1.0 / _Q) - _C) * (1.0 / _N)).astype(jnp.bfloat16)


@functools.partial(jax.jit, static_argnames=("block_b",))
def _tc_term(logits, targets, block_b=2):
    nsteps = _B // block_b
    return pl.pallas_call(
        _tc_term_body,
        grid=(nsteps,),
        in_specs=[
            pl.BlockSpec((block_b, _NCLS, _H, _W), lambda b: (b, 0, 0, 0)),
            pl.BlockSpec((block_b, _H, _W), lambda b: (b, 0, 0)),
        ],
        out_specs=pl.BlockSpec((block_b, _H, _W), lambda b: (b, 0, 0)),
        out_shape=jax.ShapeDtypeStruct((_B, _H, _W), jnp.bfloat16),
    )(logits, targets)


def _tc_dot_body(t_ref, w_ref, out_ref):
    step = pl.program_id(0)

    @pl.when(step == 0)
    def _init():
        out_ref[0, 0] = 0.0

    out_ref[0, 0] += jnp.sum(t_ref[...].astype(jnp.float32) * w_ref[...])


@functools.partial(jax.jit, static_argnames=("block_b",))
def _tc_dot(t16, w16, block_b=8):
    nsteps = _B // block_b
    return pl.pallas_call(
        _tc_dot_body,
        grid=(nsteps,),
        in_specs=[
            pl.BlockSpec((block_b, _H, _W), lambda b: (b, 0, 0)),
            pl.BlockSpec((block_b, _H, _W), lambda b: (b, 0, 0)),
        ],
        out_specs=pl.BlockSpec((1, 1), lambda b: (0, 0),
                               memory_space=pltpu.SMEM),
        out_shape=jax.ShapeDtypeStruct((1, 1), jnp.float32),
    )(t16, w16)


def kernel(logits, weight, targets, indexes):
    w16 = _sc_gather(weight.reshape(_ROWS, _H, _W), indexes)
    t16 = _tc_term(logits, targets.reshape(_B, _H, _W))
    out = _tc_dot(t16, w16)
    return out[0, 0]
